# Initial kernel scaffold; baseline (speedup 1.0000x reference)
import functools

import jax
import jax.numpy as jnp
from jax import lax
from jax.experimental import pallas as pl
from jax.experimental.pallas import tpu as pltpu
from jax.experimental.pallas import tpu_sc as plsc

NC, NS = 2, 16
NW = NC * NS
B, L, E = 16384, 200, 32
H1, NCLS = 128, 2
SPW = B // NW
G = 4
NG = SPW // G
C0 = 128
C1 = L - C0


def _pool_sc(x, table):
    mesh = plsc.VectorSubcoreMesh(core_axis_name="c", subcore_axis_name="s")

    @functools.partial(
        pl.kernel,
        out_type=jax.ShapeDtypeStruct((B, E), jnp.float32),
        mesh=mesh,
        scratch_types=[
            pltpu.VMEM((G, L), jnp.int32),
            pltpu.VMEM((G * L, E), jnp.float32),
            pltpu.VMEM((G, E), jnp.float32),
            pltpu.SemaphoreType.DMA,
        ],
    )
    def k(x_hbm, tbl_hbm, out_hbm, idx_v, rows_v, pool_v, sem):
        wid = lax.axis_index("s") * NC + lax.axis_index("c")
        base = wid * SPW

        def group_body(g, carry):
            s0 = base + g * G
            pltpu.sync_copy(x_hbm.at[pl.ds(s0, G), :], idx_v)
            descs = []
            for i in range(G):
                descs.append(pltpu.async_copy(
                    tbl_hbm.at[idx_v.at[i, pl.ds(0, C0)]],
                    rows_v.at[pl.ds(i * L, C0)], sem))
                descs.append(pltpu.async_copy(
                    tbl_hbm.at[idx_v.at[i, pl.ds(C0, C1)]],
                    rows_v.at[pl.ds(i * L + C0, C1)], sem))
            for d in descs:
                d.wait()
            for i in range(G):
                ib = i * L
                accs = []
                for k4 in range(4):
                    accs.append((rows_v[ib + k4, pl.ds(0, 16)],
                                 rows_v[ib + k4, pl.ds(16, 16)]))

                def red_body(r, a, ib=ib):
                    out = []
                    for k4 in range(4):
                        rr = ib + 4 * r + k4
                        out.append((
                            jnp.maximum(a[k4][0], rows_v[rr, pl.ds(0, 16)]),
                            jnp.maximum(a[k4][1], rows_v[rr, pl.ds(16, 16)]),
                        ))
                    return tuple(out)

                accs = lax.fori_loop(1, L // 4, red_body, tuple(accs))
                m0 = jnp.maximum(jnp.maximum(accs[0][0], accs[1][0]),
                                 jnp.maximum(accs[2][0], accs[3][0]))
                m1 = jnp.maximum(jnp.maximum(accs[0][1], accs[1][1]),
                                 jnp.maximum(accs[2][1], accs[3][1]))
                pool_v[i, pl.ds(0, 16)] = m0
                pool_v[i, pl.ds(16, 16)] = m1
            pltpu.sync_copy(pool_v, out_hbm.at[pl.ds(s0, G), :])
            return carry

        lax.fori_loop(0, NG, group_body, 0)

    return k(x, table)


def _mlp_tc(pooled, W1, b1, W2, b2):
    BLK = 2048

    def mk(p_ref, w1_ref, b1_ref, w2_ref, b2_ref, o_ref):
        h = jnp.dot(p_ref[...], w1_ref[...],
                    preferred_element_type=jnp.float32) + b1_ref[...]
        h = jnp.maximum(h, 0.0)
        o_ref[...] = jnp.dot(h, w2_ref[...],
                             preferred_element_type=jnp.float32) + b2_ref[...]

    return pl.pallas_call(
        mk,
        grid=(B // BLK,),
        in_specs=[
            pl.BlockSpec((BLK, E), lambda i: (i, 0)),
            pl.BlockSpec((E, H1), lambda i: (0, 0)),
            pl.BlockSpec((1, H1), lambda i: (0, 0)),
            pl.BlockSpec((H1, NCLS), lambda i: (0, 0)),
            pl.BlockSpec((1, NCLS), lambda i: (0, 0)),
        ],
        out_specs=pl.BlockSpec((BLK, NCLS), lambda i: (i, 0)),
        out_shape=jax.ShapeDtypeStruct((B, NCLS), jnp.float32),
    )(pooled, W1, b1.reshape(1, H1), W2, b2.reshape(1, NCLS))


def kernel(x, table, W1, b1, W2, b2):
    pooled = _pool_sc(x, table)
    return _mlp_tc(pooled, W1, b1, W2, b2)


# speedup vs baseline: 12.1021x; 12.1021x over previous
import functools

import jax
import jax.numpy as jnp
from jax import lax
from jax.experimental import pallas as pl
from jax.experimental.pallas import tpu as pltpu
from jax.experimental.pallas import tpu_sc as plsc

NC, NS = 2, 16
NW = NC * NS
B, L, E = 16384, 200, 32
H1, NCLS = 128, 2
SPW = B // NW
G = 4
NG = SPW // G
C0 = 128
C1 = L - C0


def _pool_sc(x, table):
    mesh = plsc.VectorSubcoreMesh(core_axis_name="c", subcore_axis_name="s")

    @functools.partial(
        pl.kernel,
        out_type=jax.ShapeDtypeStruct((B, E), jnp.float32),
        mesh=mesh,
        scratch_types=[
            pltpu.VMEM((G, L), jnp.int32),
            pltpu.VMEM((G * L, E), jnp.float32),
            pltpu.VMEM((G, E), jnp.float32),
            pltpu.SemaphoreType.DMA,
        ],
        compiler_params=pltpu.CompilerParams(use_tc_tiling_on_sc=False),
    )
    def k(x_hbm, tbl_hbm, out_hbm, idx_v, rows_v, pool_v, sem):
        wid = lax.axis_index("s") * NC + lax.axis_index("c")
        base = wid * SPW

        def group_body(g, carry):
            s0 = base + g * G
            pltpu.sync_copy(x_hbm.at[pl.ds(s0, G), :], idx_v)
            descs = []
            for i in range(G):
                descs.append(pltpu.async_copy(
                    tbl_hbm.at[idx_v.at[i, pl.ds(0, C0)]],
                    rows_v.at[pl.ds(i * L, C0)], sem))
                descs.append(pltpu.async_copy(
                    tbl_hbm.at[idx_v.at[i, pl.ds(C0, C1)]],
                    rows_v.at[pl.ds(i * L + C0, C1)], sem))
            for d in descs:
                d.wait()
            for i in range(G):
                ib = i * L
                accs = []
                for k4 in range(4):
                    accs.append((rows_v[ib + k4, pl.ds(0, 16)],
                                 rows_v[ib + k4, pl.ds(16, 16)]))

                def red_body(r, a, ib=ib):
                    out = []
                    for k4 in range(4):
                        rr = ib + 4 * r + k4
                        out.append((
                            jnp.maximum(a[k4][0], rows_v[rr, pl.ds(0, 16)]),
                            jnp.maximum(a[k4][1], rows_v[rr, pl.ds(16, 16)]),
                        ))
                    return tuple(out)

                accs = lax.fori_loop(1, L // 4, red_body, tuple(accs))
                m0 = jnp.maximum(jnp.maximum(accs[0][0], accs[1][0]),
                                 jnp.maximum(accs[2][0], accs[3][0]))
                m1 = jnp.maximum(jnp.maximum(accs[0][1], accs[1][1]),
                                 jnp.maximum(accs[2][1], accs[3][1]))
                pool_v[i, pl.ds(0, 16)] = m0
                pool_v[i, pl.ds(16, 16)] = m1
            pltpu.sync_copy(pool_v, out_hbm.at[pl.ds(s0, G), :])
            return carry

        lax.fori_loop(0, NG, group_body, 0)

    return k(x, table)


def _mlp_tc(pooled, W1, b1, W2, b2):
    BLK = 2048

    def mk(p_ref, w1_ref, b1_ref, w2_ref, b2_ref, o_ref):
        h = jnp.dot(p_ref[...], w1_ref[...],
                    preferred_element_type=jnp.float32) + b1_ref[...]
        h = jnp.maximum(h, 0.0)
        o_ref[...] = jnp.dot(h, w2_ref[...],
                             preferred_element_type=jnp.float32) + b2_ref[...]

    return pl.pallas_call(
        mk,
        grid=(B // BLK,),
        in_specs=[
            pl.BlockSpec((BLK, E), lambda i: (i, 0)),
            pl.BlockSpec((E, H1), lambda i: (0, 0)),
            pl.BlockSpec((1, H1), lambda i: (0, 0)),
            pl.BlockSpec((H1, NCLS), lambda i: (0, 0)),
            pl.BlockSpec((1, NCLS), lambda i: (0, 0)),
        ],
        out_specs=pl.BlockSpec((BLK, NCLS), lambda i: (i, 0)),
        out_shape=jax.ShapeDtypeStruct((B, NCLS), jnp.float32),
    )(pooled, W1, b1.reshape(1, H1), W2, b2.reshape(1, NCLS))


def kernel(x, table, W1, b1, W2, b2):
    pooled = _pool_sc(x, table)
    return _mlp_tc(pooled, W1, b1, W2, b2)


# speedup vs baseline: 26.6327x; 2.2007x over previous
import functools

import jax
import jax.numpy as jnp
from jax import lax
from jax.experimental import pallas as pl
from jax.experimental.pallas import tpu as pltpu
from jax.experimental.pallas import tpu_sc as plsc

NC, NS = 2, 16
NW = NC * NS
B, L, E = 16384, 200, 32
VOCAB = 1000000
H1, NCLS = 128, 2
SPW = B // NW
G = 4
NG = SPW // G
C0 = 128
C1 = L - C0


def _pool_sc(x, table):
    mesh = plsc.VectorSubcoreMesh(core_axis_name="c", subcore_axis_name="s")

    @functools.partial(
        pl.kernel,
        out_type=jax.ShapeDtypeStruct((B, E), jnp.float32),
        mesh=mesh,
        scratch_types=[
            pltpu.VMEM((G, L), jnp.int32),
            pltpu.VMEM((G, L), jnp.int32),
            pltpu.VMEM((G * L, E), jnp.float32),
            pltpu.VMEM((G * L, E), jnp.float32),
            pltpu.VMEM((G, E), jnp.float32),
            pltpu.SemaphoreType.DMA,
            pltpu.SemaphoreType.DMA,
        ],
        compiler_params=pltpu.CompilerParams(use_tc_tiling_on_sc=False),
    )
    def k(x_hbm, tbl_hbm, out_hbm, idx0, idx1, rows0, rows1, pool_v,
          sem0, sem1):
        wid = lax.axis_index("s") * NC + lax.axis_index("c")
        base = wid * SPW

        def fire(g, idx_v, rows_v, sem):
            s0 = base + g * G
            pltpu.sync_copy(x_hbm.at[pl.ds(s0, G), :], idx_v)
            for i in range(G):
                pltpu.async_copy(
                    tbl_hbm.at[idx_v.at[i, pl.ds(0, C0)]],
                    rows_v.at[pl.ds(i * L, C0)], sem)
                pltpu.async_copy(
                    tbl_hbm.at[idx_v.at[i, pl.ds(C0, C1)]],
                    rows_v.at[pl.ds(i * L + C0, C1)], sem)

        def process(g, rows_v, sem):
            pltpu.make_async_copy(
                tbl_hbm.at[pl.ds(0, G * L), :], rows_v, sem).wait()
            for i in range(G):
                ib = i * L
                accs = []
                for k4 in range(4):
                    accs.append((rows_v[ib + k4, pl.ds(0, 16)],
                                 rows_v[ib + k4, pl.ds(16, 16)]))

                def red_body(r, a, ib=ib):
                    out = []
                    for k4 in range(4):
                        rr = ib + 4 * r + k4
                        out.append((
                            jnp.maximum(a[k4][0], rows_v[rr, pl.ds(0, 16)]),
                            jnp.maximum(a[k4][1], rows_v[rr, pl.ds(16, 16)]),
                        ))
                    return tuple(out)

                accs = lax.fori_loop(1, L // 4, red_body, tuple(accs),
                                     unroll=2)
                m0 = jnp.maximum(jnp.maximum(accs[0][0], accs[1][0]),
                                 jnp.maximum(accs[2][0], accs[3][0]))
                m1 = jnp.maximum(jnp.maximum(accs[0][1], accs[1][1]),
                                 jnp.maximum(accs[2][1], accs[3][1]))
                pool_v[i, pl.ds(0, 16)] = m0
                pool_v[i, pl.ds(16, 16)] = m1
            pltpu.sync_copy(pool_v, out_hbm.at[pl.ds(base + g * G, G), :])

        fire(0, idx0, rows0, sem0)

        def group_body(g, carry):
            @pl.when((g & 1) == 0)
            def _():
                fire(g + 1, idx1, rows1, sem1)
                process(g, rows0, sem0)

            @pl.when((g & 1) == 1)
            def _():
                fire(g + 1, idx0, rows0, sem0)
                process(g, rows1, sem1)

            return carry

        lax.fori_loop(0, NG - 1, group_body, 0)
        if (NG - 1) % 2 == 0:
            process(NG - 1, rows0, sem0)
        else:
            process(NG - 1, rows1, sem1)

    return k(x, table)


def _mlp_tc(pooled, W1, b1, W2, b2):
    BLK = 2048

    def mk(p_ref, w1_ref, b1_ref, w2_ref, b2_ref, o_ref):
        h = jnp.dot(p_ref[...], w1_ref[...],
                    preferred_element_type=jnp.float32) + b1_ref[...]
        h = jnp.maximum(h, 0.0)
        o_ref[...] = jnp.dot(h, w2_ref[...],
                             preferred_element_type=jnp.float32) + b2_ref[...]

    return pl.pallas_call(
        mk,
        grid=(B // BLK,),
        in_specs=[
            pl.BlockSpec((BLK, E), lambda i: (i, 0)),
            pl.BlockSpec((E, H1), lambda i: (0, 0)),
            pl.BlockSpec((1, H1), lambda i: (0, 0)),
            pl.BlockSpec((H1, NCLS), lambda i: (0, 0)),
            pl.BlockSpec((1, NCLS), lambda i: (0, 0)),
        ],
        out_specs=pl.BlockSpec((BLK, NCLS), lambda i: (i, 0)),
        out_shape=jax.ShapeDtypeStruct((B, NCLS), jnp.float32),
    )(pooled, W1, b1.reshape(1, H1), W2, b2.reshape(1, NCLS))


TBLK = 8192
TGRID = (VOCAB + TBLK - 1) // TBLK
VOCAB_PAD = TGRID * TBLK
NT = TBLK // 512


def _table_to_linear(table_t):
    def tk(in_ref, o_ref):
        for q in range(NT):
            v = jnp.concatenate(
                [in_ref[:, pl.ds(512 * q + 128 * c, 128)] for c in range(4)],
                axis=0)
            o_ref[pl.ds(128 * q, 128), :] = jnp.transpose(v, (1, 0))

    return pl.pallas_call(
        tk,
        grid=(TGRID,),
        in_specs=[pl.BlockSpec((E, TBLK), lambda i: (0, i))],
        out_specs=pl.BlockSpec((TBLK // 4, 128), lambda i: (i, 0)),
        out_shape=jax.ShapeDtypeStruct((VOCAB_PAD * E // 128, 128), jnp.float32),
    )(table_t)


def _perm_idx(r):
    return (r & ~511) + ((r & 127) << 2) + ((r & 511) >> 7)


def kernel(x, table, W1, b1, W2, b2):
    table_lin = _table_to_linear(table.T)
    table2 = table_lin.reshape(VOCAB_PAD, E)
    x2 = _perm_idx(x)
    pooled = _pool_sc(x2, table2)
    return _mlp_tc(pooled, W1, b1, W2, b2)


# speedup vs baseline: 29.0937x; 1.0924x over previous
import functools

import jax
import jax.numpy as jnp
from jax import lax
from jax.experimental import pallas as pl
from jax.experimental.pallas import tpu as pltpu
from jax.experimental.pallas import tpu_sc as plsc

NC, NS = 2, 16
NW = NC * NS
B, L, E = 16384, 200, 32
VOCAB = 1000000
H1, NCLS = 128, 2
SPW = B // NW
G = 8
NG = SPW // G
C0 = 128
C1 = L - C0


def _pool_sc(x, table):
    mesh = plsc.VectorSubcoreMesh(core_axis_name="c", subcore_axis_name="s")

    @functools.partial(
        pl.kernel,
        out_type=jax.ShapeDtypeStruct((B, E), jnp.float32),
        mesh=mesh,
        scratch_types=[
            pltpu.VMEM((G, L), jnp.int32),
            pltpu.VMEM((G, L), jnp.int32),
            pltpu.VMEM((G * L, E), jnp.float32),
            pltpu.VMEM((G * L, E), jnp.float32),
            pltpu.VMEM((G, E), jnp.float32),
            pltpu.SemaphoreType.DMA,
            pltpu.SemaphoreType.DMA,
        ],
        compiler_params=pltpu.CompilerParams(use_tc_tiling_on_sc=False),
    )
    def k(x_hbm, tbl_hbm, out_hbm, idx0, idx1, rows0, rows1, pool_v,
          sem0, sem1):
        wid = lax.axis_index("s") * NC + lax.axis_index("c")
        base = wid * SPW

        def fire(g, idx_v, rows_v, sem):
            s0 = base + g * G
            pltpu.sync_copy(x_hbm.at[pl.ds(s0, G), :], idx_v)
            for i in range(G):
                pltpu.async_copy(
                    tbl_hbm.at[idx_v.at[i, pl.ds(0, C0)]],
                    rows_v.at[pl.ds(i * L, C0)], sem)
                pltpu.async_copy(
                    tbl_hbm.at[idx_v.at[i, pl.ds(C0, C1)]],
                    rows_v.at[pl.ds(i * L + C0, C1)], sem)

        def process(g, rows_v, sem):
            pltpu.make_async_copy(
                tbl_hbm.at[pl.ds(0, G * L), :], rows_v, sem).wait()
            for i in range(G):
                ib = i * L
                accs = []
                for k4 in range(4):
                    accs.append((rows_v[ib + k4, pl.ds(0, 16)],
                                 rows_v[ib + k4, pl.ds(16, 16)]))

                def red_body(r, a, ib=ib):
                    out = []
                    for k4 in range(4):
                        rr = ib + 4 * r + k4
                        out.append((
                            jnp.maximum(a[k4][0], rows_v[rr, pl.ds(0, 16)]),
                            jnp.maximum(a[k4][1], rows_v[rr, pl.ds(16, 16)]),
                        ))
                    return tuple(out)

                accs = lax.fori_loop(1, L // 4, red_body, tuple(accs),
                                     unroll=2)
                m0 = jnp.maximum(jnp.maximum(accs[0][0], accs[1][0]),
                                 jnp.maximum(accs[2][0], accs[3][0]))
                m1 = jnp.maximum(jnp.maximum(accs[0][1], accs[1][1]),
                                 jnp.maximum(accs[2][1], accs[3][1]))
                pool_v[i, pl.ds(0, 16)] = m0
                pool_v[i, pl.ds(16, 16)] = m1
            pltpu.sync_copy(pool_v, out_hbm.at[pl.ds(base + g * G, G), :])

        fire(0, idx0, rows0, sem0)

        def group_body(g, carry):
            @pl.when((g & 1) == 0)
            def _():
                fire(g + 1, idx1, rows1, sem1)
                process(g, rows0, sem0)

            @pl.when((g & 1) == 1)
            def _():
                fire(g + 1, idx0, rows0, sem0)
                process(g, rows1, sem1)

            return carry

        lax.fori_loop(0, NG - 1, group_body, 0)
        if (NG - 1) % 2 == 0:
            process(NG - 1, rows0, sem0)
        else:
            process(NG - 1, rows1, sem1)

    return k(x, table)


def _mlp_tc(pooled, W1, b1, W2, b2):
    BLK = 2048

    def mk(p_ref, w1_ref, b1_ref, w2_ref, b2_ref, o_ref):
        h = jnp.dot(p_ref[...], w1_ref[...],
                    preferred_element_type=jnp.float32) + b1_ref[...]
        h = jnp.maximum(h, 0.0)
        o_ref[...] = jnp.dot(h, w2_ref[...],
                             preferred_element_type=jnp.float32) + b2_ref[...]

    return pl.pallas_call(
        mk,
        grid=(B // BLK,),
        in_specs=[
            pl.BlockSpec((BLK, E), lambda i: (i, 0)),
            pl.BlockSpec((E, H1), lambda i: (0, 0)),
            pl.BlockSpec((1, H1), lambda i: (0, 0)),
            pl.BlockSpec((H1, NCLS), lambda i: (0, 0)),
            pl.BlockSpec((1, NCLS), lambda i: (0, 0)),
        ],
        out_specs=pl.BlockSpec((BLK, NCLS), lambda i: (i, 0)),
        out_shape=jax.ShapeDtypeStruct((B, NCLS), jnp.float32),
    )(pooled, W1, b1.reshape(1, H1), W2, b2.reshape(1, NCLS))


TBLK = 8192
TGRID = (VOCAB + TBLK - 1) // TBLK
VOCAB_PAD = TGRID * TBLK
NT = TBLK // 512


def _table_to_linear(table_t):
    def tk(in_ref, o_ref):
        for q in range(NT):
            v = jnp.concatenate(
                [in_ref[:, pl.ds(512 * q + 128 * c, 128)] for c in range(4)],
                axis=0)
            o_ref[pl.ds(128 * q, 128), :] = jnp.transpose(v, (1, 0))

    return pl.pallas_call(
        tk,
        grid=(TGRID,),
        in_specs=[pl.BlockSpec((E, TBLK), lambda i: (0, i))],
        out_specs=pl.BlockSpec((TBLK // 4, 128), lambda i: (i, 0)),
        out_shape=jax.ShapeDtypeStruct((VOCAB_PAD * E // 128, 128), jnp.float32),
    )(table_t)


def _perm_idx(r):
    return (r & ~511) + ((r & 127) << 2) + ((r & 511) >> 7)


def kernel(x, table, W1, b1, W2, b2):
    table_lin = _table_to_linear(table.T)
    table2 = table_lin.reshape(VOCAB_PAD, E)
    x2 = _perm_idx(x)
    pooled = _pool_sc(x2, table2)
    return _mlp_tc(pooled, W1, b1, W2, b2)


# speedup vs baseline: 31.7732x; 1.0921x over previous
import functools

import jax
import jax.numpy as jnp
from jax import lax
from jax.experimental import pallas as pl
from jax.experimental.pallas import tpu as pltpu
from jax.experimental.pallas import tpu_sc as plsc

NC, NS = 2, 16
NW = NC * NS
B, L, E = 16384, 200, 32
VOCAB = 1000000
H1, NCLS = 128, 2
SPW = B // NW
G = 8
NG = SPW // G
C0 = 128
C1 = L - C0


def _pool_sc(x, table):
    mesh = plsc.VectorSubcoreMesh(core_axis_name="c", subcore_axis_name="s")

    @functools.partial(
        pl.kernel,
        out_type=jax.ShapeDtypeStruct((B, E), jnp.float32),
        mesh=mesh,
        scratch_types=[
            pltpu.VMEM((G, L), jnp.int32),
            pltpu.VMEM((G, L), jnp.int32),
            pltpu.VMEM((G * L, E), jnp.float32),
            pltpu.VMEM((G * L, E), jnp.float32),
            pltpu.VMEM((G, E), jnp.float32),
            pltpu.SemaphoreType.DMA,
            pltpu.SemaphoreType.DMA,
        ],
        compiler_params=pltpu.CompilerParams(use_tc_tiling_on_sc=False),
    )
    def k(x_hbm, tbl_hbm, out_hbm, idx0, idx1, rows0, rows1, pool_v,
          sem0, sem1):
        wid = lax.axis_index("s") * NC + lax.axis_index("c")
        base = wid * SPW

        def fire(g, idx_v, rows_v, sem):
            s0 = base + g * G
            pltpu.sync_copy(x_hbm.at[pl.ds(s0, G), :], idx_v)
            for i in range(G):
                pltpu.async_copy(
                    tbl_hbm.at[idx_v.at[i, pl.ds(0, C0)]],
                    rows_v.at[pl.ds(i * L, C0)], sem)
                pltpu.async_copy(
                    tbl_hbm.at[idx_v.at[i, pl.ds(C0, C1)]],
                    rows_v.at[pl.ds(i * L + C0, C1)], sem)

        def process(g, rows_v, sem):
            pltpu.make_async_copy(
                tbl_hbm.at[pl.ds(0, G * L), :], rows_v, sem).wait()
            for i in range(G):
                ib = i * L
                accs = []
                for k4 in range(4):
                    accs.append((rows_v[ib + k4, pl.ds(0, 16)],
                                 rows_v[ib + k4, pl.ds(16, 16)]))

                def red_body(r, a, ib=ib):
                    out = []
                    for k4 in range(4):
                        rr = ib + 4 * r + k4
                        out.append((
                            jnp.maximum(a[k4][0], rows_v[rr, pl.ds(0, 16)]),
                            jnp.maximum(a[k4][1], rows_v[rr, pl.ds(16, 16)]),
                        ))
                    return tuple(out)

                accs = lax.fori_loop(1, L // 4, red_body, tuple(accs),
                                     unroll=2)
                m0 = jnp.maximum(jnp.maximum(accs[0][0], accs[1][0]),
                                 jnp.maximum(accs[2][0], accs[3][0]))
                m1 = jnp.maximum(jnp.maximum(accs[0][1], accs[1][1]),
                                 jnp.maximum(accs[2][1], accs[3][1]))
                pool_v[i, pl.ds(0, 16)] = m0
                pool_v[i, pl.ds(16, 16)] = m1
            pltpu.sync_copy(pool_v, out_hbm.at[pl.ds(base + g * G, G), :])

        fire(0, idx0, rows0, sem0)

        def group_body(g, carry):
            @pl.when((g & 1) == 0)
            def _():
                fire(g + 1, idx1, rows1, sem1)
                process(g, rows0, sem0)

            @pl.when((g & 1) == 1)
            def _():
                fire(g + 1, idx0, rows0, sem0)
                process(g, rows1, sem1)

            return carry

        lax.fori_loop(0, NG - 1, group_body, 0)
        if (NG - 1) % 2 == 0:
            process(NG - 1, rows0, sem0)
        else:
            process(NG - 1, rows1, sem1)

    return k(x, table)


def _mlp_tc(pooled, W1, b1, W2, b2):
    BLK = 2048

    def mk(p_ref, w1_ref, b1_ref, w2_ref, b2_ref, o_ref):
        h = jnp.dot(p_ref[...], w1_ref[...],
                    preferred_element_type=jnp.float32) + b1_ref[...]
        h = jnp.maximum(h, 0.0)
        o_ref[...] = jnp.dot(h, w2_ref[...],
                             preferred_element_type=jnp.float32) + b2_ref[...]

    return pl.pallas_call(
        mk,
        grid=(B // BLK,),
        in_specs=[
            pl.BlockSpec((BLK, E), lambda i: (i, 0)),
            pl.BlockSpec((E, H1), lambda i: (0, 0)),
            pl.BlockSpec((1, H1), lambda i: (0, 0)),
            pl.BlockSpec((H1, NCLS), lambda i: (0, 0)),
            pl.BlockSpec((1, NCLS), lambda i: (0, 0)),
        ],
        out_specs=pl.BlockSpec((BLK, NCLS), lambda i: (i, 0)),
        out_shape=jax.ShapeDtypeStruct((B, NCLS), jnp.float32),
    )(pooled, W1, b1.reshape(1, H1), W2, b2.reshape(1, NCLS))


TBLK = 16384
TGRID = (VOCAB + TBLK - 1) // TBLK
VOCAB_PAD = TGRID * TBLK
NT = TBLK // 512


def _table_to_linear(table_t):
    def tk(in_ref, o_ref):
        for q in range(NT):
            v = jnp.concatenate(
                [in_ref[:, pl.ds(512 * q + 128 * c, 128)] for c in range(4)],
                axis=0)
            o_ref[pl.ds(128 * q, 128), :] = jnp.transpose(v, (1, 0))

    return pl.pallas_call(
        tk,
        grid=(TGRID,),
        in_specs=[pl.BlockSpec((E, TBLK), lambda i: (0, i))],
        out_specs=pl.BlockSpec((TBLK // 4, 128), lambda i: (i, 0)),
        out_shape=jax.ShapeDtypeStruct((VOCAB_PAD * E // 128, 128), jnp.float32),
    )(table_t)


def _perm_idx(r):
    return (r & ~511) + ((r & 127) << 2) + ((r & 511) >> 7)


def kernel(x, table, W1, b1, W2, b2):
    table_lin = _table_to_linear(table.T)
    table2 = table_lin.reshape(VOCAB_PAD, E)
    x2 = _perm_idx(x)
    pooled = _pool_sc(x2, table2)
    return _mlp_tc(pooled, W1, b1, W2, b2)


# speedup vs baseline: 32.9269x; 1.0363x over previous
import functools

import jax
import jax.numpy as jnp
from jax import lax
from jax.experimental import pallas as pl
from jax.experimental.pallas import tpu as pltpu
from jax.experimental.pallas import tpu_sc as plsc

NC, NS = 2, 16
NW = NC * NS
B, L, E = 16384, 200, 32
VOCAB = 1000000
H1, NCLS = 128, 2
SPW = B // NW
G = 8
NG = SPW // G
C0 = 128
C1 = L - C0


def _pool_sc(x, table):
    mesh = plsc.VectorSubcoreMesh(core_axis_name="c", subcore_axis_name="s")

    @functools.partial(
        pl.kernel,
        out_type=jax.ShapeDtypeStruct((B, E), jnp.float32),
        mesh=mesh,
        scratch_types=[
            pltpu.VMEM((G, L), jnp.int32),
            pltpu.VMEM((G, L), jnp.int32),
            pltpu.VMEM((G * L, E), jnp.float32),
            pltpu.VMEM((G * L, E), jnp.float32),
            pltpu.VMEM((G, E), jnp.float32),
            pltpu.SemaphoreType.DMA,
            pltpu.SemaphoreType.DMA,
        ],
        compiler_params=pltpu.CompilerParams(use_tc_tiling_on_sc=False),
    )
    def k(x_hbm, tbl_hbm, out_hbm, idx0, idx1, rows0, rows1, pool_v,
          sem0, sem1):
        wid = lax.axis_index("s") * NC + lax.axis_index("c")
        base = wid * SPW

        def fire(g, idx_v, rows_v, sem):
            s0 = base + g * G
            pltpu.sync_copy(x_hbm.at[pl.ds(s0, G), :], idx_v)
            for i in range(G):
                pltpu.async_copy(
                    tbl_hbm.at[idx_v.at[i, pl.ds(0, C0)]],
                    rows_v.at[pl.ds(i * L, C0)], sem)
                pltpu.async_copy(
                    tbl_hbm.at[idx_v.at[i, pl.ds(C0, C1)]],
                    rows_v.at[pl.ds(i * L + C0, C1)], sem)

        def process(g, rows_v, sem):
            pltpu.make_async_copy(
                tbl_hbm.at[pl.ds(0, G * L), :], rows_v, sem).wait()
            for i in range(G):
                ib = i * L
                accs = []
                for k4 in range(4):
                    accs.append((rows_v[ib + k4, pl.ds(0, 16)],
                                 rows_v[ib + k4, pl.ds(16, 16)]))

                def red_body(r, a, ib=ib):
                    out = []
                    for k4 in range(4):
                        rr = ib + 4 * r + k4
                        out.append((
                            jnp.maximum(a[k4][0], rows_v[rr, pl.ds(0, 16)]),
                            jnp.maximum(a[k4][1], rows_v[rr, pl.ds(16, 16)]),
                        ))
                    return tuple(out)

                accs = lax.fori_loop(1, L // 4, red_body, tuple(accs),
                                     unroll=2)
                m0 = jnp.maximum(jnp.maximum(accs[0][0], accs[1][0]),
                                 jnp.maximum(accs[2][0], accs[3][0]))
                m1 = jnp.maximum(jnp.maximum(accs[0][1], accs[1][1]),
                                 jnp.maximum(accs[2][1], accs[3][1]))
                pool_v[i, pl.ds(0, 16)] = m0
                pool_v[i, pl.ds(16, 16)] = m1
            pltpu.sync_copy(pool_v, out_hbm.at[pl.ds(base + g * G, G), :])

        fire(0, idx0, rows0, sem0)

        def group_body(g, carry):
            @pl.when((g & 1) == 0)
            def _():
                fire(g + 1, idx1, rows1, sem1)
                process(g, rows0, sem0)

            @pl.when((g & 1) == 1)
            def _():
                fire(g + 1, idx0, rows0, sem0)
                process(g, rows1, sem1)

            return carry

        lax.fori_loop(0, NG - 1, group_body, 0)
        if (NG - 1) % 2 == 0:
            process(NG - 1, rows0, sem0)
        else:
            process(NG - 1, rows1, sem1)

    return k(x, table)


def _mlp_tc(pooled, W1, b1, W2, b2):
    BLK = 2048

    def mk(p_ref, w1_ref, b1_ref, w2_ref, b2_ref, o_ref):
        h = jnp.dot(p_ref[...], w1_ref[...],
                    preferred_element_type=jnp.float32) + b1_ref[...]
        h = jnp.maximum(h, 0.0)
        o_ref[...] = jnp.dot(h, w2_ref[...],
                             preferred_element_type=jnp.float32) + b2_ref[...]

    return pl.pallas_call(
        mk,
        grid=(B // BLK,),
        in_specs=[
            pl.BlockSpec((BLK, E), lambda i: (i, 0)),
            pl.BlockSpec((E, H1), lambda i: (0, 0)),
            pl.BlockSpec((1, H1), lambda i: (0, 0)),
            pl.BlockSpec((H1, NCLS), lambda i: (0, 0)),
            pl.BlockSpec((1, NCLS), lambda i: (0, 0)),
        ],
        out_specs=pl.BlockSpec((BLK, NCLS), lambda i: (i, 0)),
        out_shape=jax.ShapeDtypeStruct((B, NCLS), jnp.float32),
    )(pooled, W1, b1.reshape(1, H1), W2, b2.reshape(1, NCLS))


TBLK = 32768
TGRID = (VOCAB + TBLK - 1) // TBLK
VOCAB_PAD = TGRID * TBLK
NT = TBLK // 512


def _table_to_linear(table_t):
    def tk(in_ref, o_ref):
        for q in range(NT):
            v = jnp.concatenate(
                [in_ref[:, pl.ds(512 * q + 128 * c, 128)] for c in range(4)],
                axis=0)
            o_ref[pl.ds(128 * q, 128), :] = jnp.transpose(v, (1, 0))

    return pl.pallas_call(
        tk,
        grid=(TGRID,),
        in_specs=[pl.BlockSpec((E, TBLK), lambda i: (0, i))],
        out_specs=pl.BlockSpec((TBLK // 4, 128), lambda i: (i, 0)),
        out_shape=jax.ShapeDtypeStruct((VOCAB_PAD * E // 128, 128), jnp.float32),
    )(table_t)


def _perm_idx(r):
    return (r & ~511) + ((r & 127) << 2) + ((r & 511) >> 7)


def kernel(x, table, W1, b1, W2, b2):
    table_lin = _table_to_linear(table.T)
    table2 = table_lin.reshape(VOCAB_PAD, E)
    x2 = _perm_idx(x)
    pooled = _pool_sc(x2, table2)
    return _mlp_tc(pooled, W1, b1, W2, b2)


# speedup vs baseline: 33.1175x; 1.0058x over previous
import functools

import jax
import jax.numpy as jnp
from jax import lax
from jax.experimental import pallas as pl
from jax.experimental.pallas import tpu as pltpu
from jax.experimental.pallas import tpu_sc as plsc

NC, NS = 2, 16
NW = NC * NS
B, L, E = 16384, 200, 32
VOCAB = 1000000
H1, NCLS = 128, 2
SPW = B // NW
G = 8
NG = SPW // G
C0 = 128
C1 = L - C0


def _pool_sc(x, table):
    mesh = plsc.VectorSubcoreMesh(core_axis_name="c", subcore_axis_name="s")

    @functools.partial(
        pl.kernel,
        out_type=jax.ShapeDtypeStruct((B, E), jnp.float32),
        mesh=mesh,
        scratch_types=[
            pltpu.VMEM((G, L), jnp.int32),
            pltpu.VMEM((G, L), jnp.int32),
            pltpu.VMEM((G * L, E), jnp.float32),
            pltpu.VMEM((G * L, E), jnp.float32),
            pltpu.VMEM((G, E), jnp.float32),
            pltpu.SemaphoreType.DMA,
            pltpu.SemaphoreType.DMA,
        ],
        compiler_params=pltpu.CompilerParams(use_tc_tiling_on_sc=False),
    )
    def k(x_hbm, tbl_hbm, out_hbm, idx0, idx1, rows0, rows1, pool_v,
          sem0, sem1):
        wid = lax.axis_index("s") * NC + lax.axis_index("c")
        base = wid * SPW

        def fire(g, idx_v, rows_v, sem):
            s0 = base + g * G
            pltpu.sync_copy(x_hbm.at[pl.ds(s0, G), :], idx_v)
            for i in range(G):
                pltpu.async_copy(
                    tbl_hbm.at[idx_v.at[i, pl.ds(0, C0)]],
                    rows_v.at[pl.ds(i * L, C0)], sem)
                pltpu.async_copy(
                    tbl_hbm.at[idx_v.at[i, pl.ds(C0, C1)]],
                    rows_v.at[pl.ds(i * L + C0, C1)], sem)

        def process(g, rows_v, sem):
            pltpu.make_async_copy(
                tbl_hbm.at[pl.ds(0, G * L), :], rows_v, sem).wait()
            for i in range(G):
                ib = i * L
                accs = []
                for k4 in range(4):
                    accs.append((rows_v[ib + k4, pl.ds(0, 16)],
                                 rows_v[ib + k4, pl.ds(16, 16)]))

                def red_body(r, a, ib=ib):
                    out = []
                    for k4 in range(4):
                        rr = ib + 4 * r + k4
                        out.append((
                            jnp.maximum(a[k4][0], rows_v[rr, pl.ds(0, 16)]),
                            jnp.maximum(a[k4][1], rows_v[rr, pl.ds(16, 16)]),
                        ))
                    return tuple(out)

                accs = lax.fori_loop(1, L // 4, red_body, tuple(accs),
                                     unroll=2)
                m0 = jnp.maximum(jnp.maximum(accs[0][0], accs[1][0]),
                                 jnp.maximum(accs[2][0], accs[3][0]))
                m1 = jnp.maximum(jnp.maximum(accs[0][1], accs[1][1]),
                                 jnp.maximum(accs[2][1], accs[3][1]))
                pool_v[i, pl.ds(0, 16)] = m0
                pool_v[i, pl.ds(16, 16)] = m1
            pltpu.sync_copy(pool_v, out_hbm.at[pl.ds(base + g * G, G), :])

        fire(0, idx0, rows0, sem0)

        def group_body(g, carry):
            @pl.when((g & 1) == 0)
            def _():
                fire(g + 1, idx1, rows1, sem1)
                process(g, rows0, sem0)

            @pl.when((g & 1) == 1)
            def _():
                fire(g + 1, idx0, rows0, sem0)
                process(g, rows1, sem1)

            return carry

        lax.fori_loop(0, NG - 1, group_body, 0)
        if (NG - 1) % 2 == 0:
            process(NG - 1, rows0, sem0)
        else:
            process(NG - 1, rows1, sem1)

    return k(x, table)


def _mlp_tc(pooled, W1, b1, W2, b2):
    BLK = 2048

    def mk(p_ref, w1_ref, b1_ref, w2_ref, b2_ref, o_ref):
        h = jnp.dot(p_ref[...], w1_ref[...],
                    preferred_element_type=jnp.float32) + b1_ref[...]
        h = jnp.maximum(h, 0.0)
        o_ref[...] = jnp.dot(h, w2_ref[...],
                             preferred_element_type=jnp.float32) + b2_ref[...]

    return pl.pallas_call(
        mk,
        grid=(B // BLK,),
        in_specs=[
            pl.BlockSpec((BLK, E), lambda i: (i, 0)),
            pl.BlockSpec((E, H1), lambda i: (0, 0)),
            pl.BlockSpec((1, H1), lambda i: (0, 0)),
            pl.BlockSpec((H1, NCLS), lambda i: (0, 0)),
            pl.BlockSpec((1, NCLS), lambda i: (0, 0)),
        ],
        out_specs=pl.BlockSpec((BLK, NCLS), lambda i: (i, 0)),
        out_shape=jax.ShapeDtypeStruct((B, NCLS), jnp.float32),
    )(pooled, W1, b1.reshape(1, H1), W2, b2.reshape(1, NCLS))


TBLK = 65536
TGRID = (VOCAB + TBLK - 1) // TBLK
VOCAB_PAD = TGRID * TBLK
NT = TBLK // 512


def _table_to_linear(table_t):
    def tk(in_ref, o_ref):
        for q in range(NT):
            v = jnp.concatenate(
                [in_ref[:, pl.ds(512 * q + 128 * c, 128)] for c in range(4)],
                axis=0)
            o_ref[pl.ds(128 * q, 128), :] = jnp.transpose(v, (1, 0))

    return pl.pallas_call(
        tk,
        grid=(TGRID,),
        in_specs=[pl.BlockSpec((E, TBLK), lambda i: (0, i))],
        out_specs=pl.BlockSpec((TBLK // 4, 128), lambda i: (i, 0)),
        out_shape=jax.ShapeDtypeStruct((VOCAB_PAD * E // 128, 128), jnp.float32),
    )(table_t)


def _perm_idx(r):
    return (r & ~511) + ((r & 127) << 2) + ((r & 511) >> 7)


def kernel(x, table, W1, b1, W2, b2):
    table_lin = _table_to_linear(table.T)
    table2 = table_lin.reshape(VOCAB_PAD, E)
    x2 = _perm_idx(x)
    pooled = _pool_sc(x2, table2)
    return _mlp_tc(pooled, W1, b1, W2, b2)


# speedup vs baseline: 36.4537x; 1.1007x over previous
import functools

import jax
import jax.numpy as jnp
from jax import lax
from jax.experimental import pallas as pl
from jax.experimental.pallas import tpu as pltpu
from jax.experimental.pallas import tpu_sc as plsc

NC, NS = 2, 16
NW = NC * NS
B, L, E = 16384, 200, 32
VOCAB = 1000000
H1, NCLS = 128, 2
SPW = B // NW
G = 8
NG = SPW // G
C0 = 128
C1 = L - C0


def _pool_sc(xc, table):
    mesh = plsc.VectorSubcoreMesh(core_axis_name="c", subcore_axis_name="s")

    @functools.partial(
        pl.kernel,
        out_type=jax.ShapeDtypeStruct((B, E), jnp.float32),
        mesh=mesh,
        scratch_types=[
            pltpu.VMEM((2 * G, 128), jnp.int32),
            pltpu.VMEM((2 * G, 128), jnp.int32),
            pltpu.VMEM((G * L, E), jnp.float32),
            pltpu.VMEM((G * L, E), jnp.float32),
            pltpu.VMEM((G, E), jnp.float32),
            pltpu.SemaphoreType.DMA,
            pltpu.SemaphoreType.DMA,
        ],
        compiler_params=pltpu.CompilerParams(use_tc_tiling_on_sc=False),
    )
    def k(x_hbm, tbl_hbm, out_hbm, idx0, idx1, rows0, rows1, pool_v,
          sem0, sem1):
        wid = lax.axis_index("s") * NC + lax.axis_index("c")
        base = wid * SPW

        def fire(g, idx_v, rows_v, sem):
            pltpu.sync_copy(x_hbm.at[base // G + g], idx_v)
            for i in range(G):
                pltpu.async_copy(
                    tbl_hbm.at[idx_v.at[i, :]],
                    rows_v.at[pl.ds(i * L, C0)], sem)
                pltpu.async_copy(
                    tbl_hbm.at[idx_v.at[G + i, pl.ds(128 - C1, C1)]],
                    rows_v.at[pl.ds(i * L + C0, C1)], sem)

        def process(g, rows_v, sem):
            pltpu.make_async_copy(
                tbl_hbm.at[pl.ds(0, G * L), :], rows_v, sem).wait()
            for i in range(G):
                ib = i * L
                accs = []
                for k4 in range(4):
                    accs.append((rows_v[ib + k4, pl.ds(0, 16)],
                                 rows_v[ib + k4, pl.ds(16, 16)]))

                def red_body(r, a, ib=ib):
                    out = []
                    for k4 in range(4):
                        rr = ib + 4 * r + k4
                        out.append((
                            jnp.maximum(a[k4][0], rows_v[rr, pl.ds(0, 16)]),
                            jnp.maximum(a[k4][1], rows_v[rr, pl.ds(16, 16)]),
                        ))
                    return tuple(out)

                accs = lax.fori_loop(1, L // 4, red_body, tuple(accs),
                                     unroll=2)
                m0 = jnp.maximum(jnp.maximum(accs[0][0], accs[1][0]),
                                 jnp.maximum(accs[2][0], accs[3][0]))
                m1 = jnp.maximum(jnp.maximum(accs[0][1], accs[1][1]),
                                 jnp.maximum(accs[2][1], accs[3][1]))
                pool_v[i, pl.ds(0, 16)] = m0
                pool_v[i, pl.ds(16, 16)] = m1
            pltpu.sync_copy(pool_v, out_hbm.at[pl.ds(base + g * G, G), :])

        fire(0, idx0, rows0, sem0)

        def group_body(g, carry):
            @pl.when((g & 1) == 0)
            def _():
                fire(g + 1, idx1, rows1, sem1)
                process(g, rows0, sem0)

            @pl.when((g & 1) == 1)
            def _():
                fire(g + 1, idx0, rows0, sem0)
                process(g, rows1, sem1)

            return carry

        lax.fori_loop(0, NG - 1, group_body, 0)
        if (NG - 1) % 2 == 0:
            process(NG - 1, rows0, sem0)
        else:
            process(NG - 1, rows1, sem1)

    return k(xc, table)


def _mlp_tc(pooled, W1, b1, W2, b2):
    BLK = 2048

    def mk(p_ref, w1_ref, b1_ref, w2_ref, b2_ref, o_ref):
        h = jnp.dot(p_ref[...], w1_ref[...],
                    preferred_element_type=jnp.float32) + b1_ref[...]
        h = jnp.maximum(h, 0.0)
        o_ref[...] = jnp.dot(h, w2_ref[...],
                             preferred_element_type=jnp.float32) + b2_ref[...]

    return pl.pallas_call(
        mk,
        grid=(B // BLK,),
        in_specs=[
            pl.BlockSpec((BLK, E), lambda i: (i, 0)),
            pl.BlockSpec((E, H1), lambda i: (0, 0)),
            pl.BlockSpec((1, H1), lambda i: (0, 0)),
            pl.BlockSpec((H1, NCLS), lambda i: (0, 0)),
            pl.BlockSpec((1, NCLS), lambda i: (0, 0)),
        ],
        out_specs=pl.BlockSpec((BLK, NCLS), lambda i: (i, 0)),
        out_shape=jax.ShapeDtypeStruct((B, NCLS), jnp.float32),
    )(pooled, W1, b1.reshape(1, H1), W2, b2.reshape(1, NCLS))


TBLK = 65536
TGRID = (VOCAB + TBLK - 1) // TBLK
VOCAB_PAD = TGRID * TBLK
NT = TBLK // 512


def _table_to_linear(table_t):
    def tk(in_ref, o_ref):
        for q in range(NT):
            v = jnp.concatenate(
                [in_ref[:, pl.ds(512 * q + 128 * c, 128)] for c in range(4)],
                axis=0)
            o_ref[pl.ds(128 * q, 128), :] = jnp.transpose(v, (1, 0))

    return pl.pallas_call(
        tk,
        grid=(TGRID,),
        in_specs=[pl.BlockSpec((E, TBLK), lambda i: (0, i))],
        out_specs=pl.BlockSpec((TBLK // 4, 128), lambda i: (i, 0)),
        out_shape=jax.ShapeDtypeStruct((VOCAB_PAD * E // 128, 128), jnp.float32),
    )(table_t)


def _perm_idx(r):
    return (r & ~511) + ((r & 127) << 2) + ((r & 511) >> 7)


SBLK = 2048
SGRID = B // SBLK


def _x_to_linear(x_t):
    def xk(in_ref, oa_ref):
        t = _perm_idx(in_ref[...])
        for q in range(SBLK // 128):
            m = slice(128 * q, 128 * (q + 1))
            a = jnp.transpose(t[0:128, m], (1, 0))
            bpart = jnp.transpose(t[L - 128:L, m], (1, 0))
            oa_ref[pl.ds(16 * q, 16), 0, :, :] = a.reshape(16, 8, 128)
            oa_ref[pl.ds(16 * q, 16), 1, :, :] = bpart.reshape(16, 8, 128)

    return pl.pallas_call(
        xk,
        grid=(SGRID,),
        in_specs=[pl.BlockSpec((L, SBLK), lambda i: (0, i))],
        out_specs=pl.BlockSpec((SBLK // 8, 2, 8, 128), lambda i: (i, 0, 0, 0)),
        out_shape=jax.ShapeDtypeStruct((B // 8, 2, 8, 128), jnp.int32),
    )(x_t)


def kernel(x, table, W1, b1, W2, b2):
    table_lin = _table_to_linear(table.T)
    table2 = table_lin.reshape(VOCAB_PAD, E)
    xc = _x_to_linear(x.T).reshape(B // 8, 16, 128)
    pooled = _pool_sc(xc, table2)
    return _mlp_tc(pooled, W1, b1, W2, b2)


# speedup vs baseline: 36.7202x; 1.0073x over previous
import functools

import jax
import jax.numpy as jnp
from jax import lax
from jax.experimental import pallas as pl
from jax.experimental.pallas import tpu as pltpu
from jax.experimental.pallas import tpu_sc as plsc

NC, NS = 2, 16
NW = NC * NS
B, L, E = 16384, 200, 32
VOCAB = 1000000
H1, NCLS = 128, 2
SPW = B // NW
G = 8
NG = SPW // G
C0 = 128
C1 = L - C0


def _pool_sc(xc, table):
    mesh = plsc.VectorSubcoreMesh(core_axis_name="c", subcore_axis_name="s")

    @functools.partial(
        pl.kernel,
        out_type=jax.ShapeDtypeStruct((B, E), jnp.float32),
        mesh=mesh,
        scratch_types=[
            pltpu.VMEM((2 * G, 128), jnp.int32),
            pltpu.VMEM((2 * G, 128), jnp.int32),
            pltpu.VMEM((G * L, E), jnp.float32),
            pltpu.VMEM((G * L, E), jnp.float32),
            pltpu.VMEM((G, E), jnp.float32),
            pltpu.VMEM((G, E), jnp.float32),
            pltpu.SemaphoreType.DMA,
            pltpu.SemaphoreType.DMA,
            pltpu.SemaphoreType.DMA,
            pltpu.SemaphoreType.DMA,
            pltpu.SemaphoreType.DMA,
            pltpu.SemaphoreType.DMA,
        ],
        compiler_params=pltpu.CompilerParams(use_tc_tiling_on_sc=False),
    )
    def k(x_hbm, tbl_hbm, out_hbm, idx0, idx1, rows0, rows1, pool0, pool1,
          sem0, sem1, isem0, isem1, psem0, psem1):
        wid = lax.axis_index("s") * NC + lax.axis_index("c")
        base = wid * SPW

        def stage(g, idx_v, isem):
            pltpu.async_copy(x_hbm.at[base // G + g], idx_v, isem)

        def fire(g, idx_v, rows_v, sem, isem):
            pltpu.make_async_copy(x_hbm.at[0], idx_v, isem).wait()
            for i in range(G):
                pltpu.async_copy(
                    tbl_hbm.at[idx_v.at[i, :]],
                    rows_v.at[pl.ds(i * L, C0)], sem)
                pltpu.async_copy(
                    tbl_hbm.at[idx_v.at[G + i, pl.ds(128 - C1, C1)]],
                    rows_v.at[pl.ds(i * L + C0, C1)], sem)

        def process(g, rows_v, sem, pool_v, psem):
            pltpu.make_async_copy(
                tbl_hbm.at[pl.ds(0, G * L), :], rows_v, sem).wait()

            @pl.when(g >= 2)
            def _():
                pltpu.make_async_copy(
                    pool_v, out_hbm.at[pl.ds(0, G), :], psem).wait()

            for i in range(G):
                ib = i * L
                accs = []
                for k4 in range(4):
                    accs.append((rows_v[ib + k4, pl.ds(0, 16)],
                                 rows_v[ib + k4, pl.ds(16, 16)]))

                def red_body(r, a, ib=ib):
                    out = []
                    for k4 in range(4):
                        rr = ib + 4 * r + k4
                        out.append((
                            jnp.maximum(a[k4][0], rows_v[rr, pl.ds(0, 16)]),
                            jnp.maximum(a[k4][1], rows_v[rr, pl.ds(16, 16)]),
                        ))
                    return tuple(out)

                accs = lax.fori_loop(1, L // 4, red_body, tuple(accs),
                                     unroll=2)
                m0 = jnp.maximum(jnp.maximum(accs[0][0], accs[1][0]),
                                 jnp.maximum(accs[2][0], accs[3][0]))
                m1 = jnp.maximum(jnp.maximum(accs[0][1], accs[1][1]),
                                 jnp.maximum(accs[2][1], accs[3][1]))
                pool_v[i, pl.ds(0, 16)] = m0
                pool_v[i, pl.ds(16, 16)] = m1
            pltpu.async_copy(
                pool_v, out_hbm.at[pl.ds(base + g * G, G), :], psem)

        stage(0, idx0, isem0)
        fire(0, idx0, rows0, sem0, isem0)
        stage(1, idx1, isem1)

        def group_body(g, carry):
            @pl.when((g & 1) == 0)
            def _():
                fire(g + 1, idx1, rows1, sem1, isem1)
                process(g, rows0, sem0, pool0, psem0)

                @pl.when(g + 2 < NG)
                def _():
                    stage(g + 2, idx0, isem0)

            @pl.when((g & 1) == 1)
            def _():
                fire(g + 1, idx0, rows0, sem0, isem0)
                process(g, rows1, sem1, pool1, psem1)

                @pl.when(g + 2 < NG)
                def _():
                    stage(g + 2, idx1, isem1)

            return carry

        lax.fori_loop(0, NG - 1, group_body, 0)
        if (NG - 1) % 2 == 0:
            process(NG - 1, rows0, sem0, pool0, psem0)
        else:
            process(NG - 1, rows1, sem1, pool1, psem1)
        pltpu.make_async_copy(pool0, out_hbm.at[pl.ds(0, G), :], psem0).wait()
        pltpu.make_async_copy(pool1, out_hbm.at[pl.ds(0, G), :], psem1).wait()

    return k(xc, table)


def _mlp_tc(pooled, W1, b1, W2, b2):
    BLK = 2048

    def mk(p_ref, w1_ref, b1_ref, w2_ref, b2_ref, o_ref):
        h = jnp.dot(p_ref[...], w1_ref[...],
                    preferred_element_type=jnp.float32) + b1_ref[...]
        h = jnp.maximum(h, 0.0)
        o_ref[...] = jnp.dot(h, w2_ref[...],
                             preferred_element_type=jnp.float32) + b2_ref[...]

    return pl.pallas_call(
        mk,
        grid=(B // BLK,),
        in_specs=[
            pl.BlockSpec((BLK, E), lambda i: (i, 0)),
            pl.BlockSpec((E, H1), lambda i: (0, 0)),
            pl.BlockSpec((1, H1), lambda i: (0, 0)),
            pl.BlockSpec((H1, NCLS), lambda i: (0, 0)),
            pl.BlockSpec((1, NCLS), lambda i: (0, 0)),
        ],
        out_specs=pl.BlockSpec((BLK, NCLS), lambda i: (i, 0)),
        out_shape=jax.ShapeDtypeStruct((B, NCLS), jnp.float32),
    )(pooled, W1, b1.reshape(1, H1), W2, b2.reshape(1, NCLS))


TBLK = 65536
TGRID = (VOCAB + TBLK - 1) // TBLK
VOCAB_PAD = TGRID * TBLK
NT = TBLK // 512


def _table_to_linear(table_t):
    def tk(in_ref, o_ref):
        for q in range(NT):
            v = jnp.concatenate(
                [in_ref[:, pl.ds(512 * q + 128 * c, 128)] for c in range(4)],
                axis=0)
            o_ref[pl.ds(128 * q, 128), :] = jnp.transpose(v, (1, 0))

    return pl.pallas_call(
        tk,
        grid=(TGRID,),
        in_specs=[pl.BlockSpec((E, TBLK), lambda i: (0, i))],
        out_specs=pl.BlockSpec((TBLK // 4, 128), lambda i: (i, 0)),
        out_shape=jax.ShapeDtypeStruct((VOCAB_PAD * E // 128, 128), jnp.float32),
    )(table_t)


def _perm_idx(r):
    return (r & ~511) + ((r & 127) << 2) + ((r & 511) >> 7)


SBLK = 2048
SGRID = B // SBLK


def _x_to_linear(x_t):
    def xk(in_ref, oa_ref):
        t = _perm_idx(in_ref[...])
        for q in range(SBLK // 128):
            m = slice(128 * q, 128 * (q + 1))
            a = jnp.transpose(t[0:128, m], (1, 0))
            bpart = jnp.transpose(t[L - 128:L, m], (1, 0))
            oa_ref[pl.ds(16 * q, 16), 0, :, :] = a.reshape(16, 8, 128)
            oa_ref[pl.ds(16 * q, 16), 1, :, :] = bpart.reshape(16, 8, 128)

    return pl.pallas_call(
        xk,
        grid=(SGRID,),
        in_specs=[pl.BlockSpec((L, SBLK), lambda i: (0, i))],
        out_specs=pl.BlockSpec((SBLK // 8, 2, 8, 128), lambda i: (i, 0, 0, 0)),
        out_shape=jax.ShapeDtypeStruct((B // 8, 2, 8, 128), jnp.int32),
    )(x_t)


def kernel(x, table, W1, b1, W2, b2):
    table_lin = _table_to_linear(table.T)
    table2 = table_lin.reshape(VOCAB_PAD, E)
    xc = _x_to_linear(x.T).reshape(B // 8, 16, 128)
    pooled = _pool_sc(xc, table2)
    return _mlp_tc(pooled, W1, b1, W2, b2)
